# baseline (speedup 1.0000x reference)
import jax
import jax.numpy as jnp
from jax.experimental import pallas as pl
from jax.experimental.pallas import tpu as pltpu

_S = 2048
_W = _S // 4
_IMG_BEGIN, _IMG_END = 128, 1152
_PAD_BEGIN_ENDS = ((0, 80), (0, 100), (0, 110), (0, 0))
_ONES = 0x01010101


def _pat_lt(n):
    c3 = jnp.clip(n, 0, 3)
    base = (jax.lax.shift_left(jnp.int32(1), c3 * 8) - 1) & jnp.int32(_ONES)
    return jnp.where(n >= 4, jnp.int32(_ONES), base)


def _interval_words(lo, hi, j4):
    return _pat_lt(hi - j4) & ~_pat_lt(lo - j4)


def _mask_kernel(pads_ref, out_ref):
    b = pl.program_id(0)
    pe = pads_ref[b, 1]

    nh = _IMG_BEGIN
    q_h = jax.lax.broadcasted_iota(jnp.int32, (nh, 1), 0)
    j4_h = jax.lax.broadcasted_iota(jnp.int32, (nh, _W), 1) * 4
    out_ref[0, 0:nh, :] = _interval_words(jnp.minimum(q_h, pe), q_h + 1, j4_h)

    j4_row = jax.lax.broadcasted_iota(jnp.int32, (1, _W), 1) * 4
    row = _interval_words(jnp.broadcast_to(pe, (1, 1)), _IMG_END, j4_row)
    out_ref[0, _IMG_BEGIN:_IMG_END, :] = jnp.broadcast_to(
        row, (_IMG_END - _IMG_BEGIN, _W)
    )

    nt = _S - _IMG_END
    q_t = _IMG_END + jax.lax.broadcasted_iota(jnp.int32, (nt, 1), 0)
    j4_t = jax.lax.broadcasted_iota(jnp.int32, (nt, _W), 1) * 4
    lo_t = _pat_lt(pe - j4_row)
    out_ref[0, _IMG_END:_S, :] = _pat_lt(q_t + 1 - j4_t) & ~lo_t


def kernel(sequence):
    B, S = sequence.shape
    pads = jnp.asarray(_PAD_BEGIN_ENDS, dtype=jnp.int32)
    packed = pl.pallas_call(
        _mask_kernel,
        grid=(B,),
        in_specs=[pl.BlockSpec(memory_space=pltpu.SMEM)],
        out_specs=pl.BlockSpec((1, S, S // 4), lambda b: (b, 0, 0)),
        out_shape=jax.ShapeDtypeStruct((B, S, S // 4), jnp.int32),
        compiler_params=pltpu.CompilerParams(
            dimension_semantics=("parallel",),
        ),
    )(pads)
    bytes_ = jax.lax.bitcast_convert_type(packed, jnp.uint8)
    return bytes_.reshape(B, S, S).astype(jnp.bool_)


# speedup vs baseline: 7.1555x; 7.1555x over previous
import jax
import jax.numpy as jnp
from jax.experimental import pallas as pl
from jax.experimental.pallas import tpu as pltpu

_S = 2048
_IMG_BEGIN, _IMG_END = 128, 1152
_PAD_BEGIN_ENDS = ((0, 80), (0, 100), (0, 110), (0, 0))
_ONES = 0x01010101


def _mask_kernel(pads_ref, out_ref):
    b = pl.program_id(0)
    pe = pads_ref[b, 1]

    nh = _IMG_BEGIN // 4
    r_h = jax.lax.broadcasted_iota(jnp.int32, (nh, 1), 0) * 4
    kv_h = jax.lax.broadcasted_iota(jnp.int32, (nh, _S), 1)
    w_h = jnp.zeros((nh, _S), jnp.int32)
    for k in range(4):
        qk = r_h + k
        lo = jnp.minimum(qk, pe)
        m = (kv_h >= lo) & (kv_h <= qk)
        w_h = w_h | jnp.where(m, jnp.int32(1 << (8 * k)), 0)
    out_ref[0, 0:_IMG_BEGIN, :] = pltpu.bitcast(w_h, jnp.int8)

    nc = (_IMG_END - _IMG_BEGIN) // 4
    kv_c = jax.lax.broadcasted_iota(jnp.int32, (nc, _S), 1)
    w_c = jnp.where((kv_c >= pe) & (kv_c < _IMG_END), jnp.int32(_ONES), 0)
    out_ref[0, _IMG_BEGIN:_IMG_END, :] = pltpu.bitcast(w_c, jnp.int8)

    nt = (_S - _IMG_END) // 4
    r_t = _IMG_END + jax.lax.broadcasted_iota(jnp.int32, (nt, 1), 0) * 4
    kv_t = jax.lax.broadcasted_iota(jnp.int32, (nt, _S), 1)
    d = kv_t - r_t
    s = jnp.minimum(jnp.maximum(d, 0), 3) * 8
    pat = jax.lax.shift_left(jnp.int32(_ONES), s)
    w_t = jnp.where((d <= 3) & (kv_t >= pe), pat, 0)
    out_ref[0, _IMG_END:_S, :] = pltpu.bitcast(w_t, jnp.int8)


def kernel(sequence):
    B, S = sequence.shape
    pads = jnp.asarray(_PAD_BEGIN_ENDS, dtype=jnp.int32)
    packed = pl.pallas_call(
        _mask_kernel,
        grid=(B,),
        in_specs=[pl.BlockSpec(memory_space=pltpu.SMEM)],
        out_specs=pl.BlockSpec((1, S, S), lambda b: (b, 0, 0)),
        out_shape=jax.ShapeDtypeStruct((B, S, S), jnp.int8),
        compiler_params=pltpu.CompilerParams(
            dimension_semantics=("parallel",),
        ),
    )(pads)
    return packed.astype(jnp.bool_)


# speedup vs baseline: 7.1556x; 1.0000x over previous
import jax
import jax.numpy as jnp
from jax.experimental import pallas as pl
from jax.experimental.pallas import tpu as pltpu

_S = 2048
_IMG_BEGIN, _IMG_END = 128, 1152
_PAD_BEGIN_ENDS = ((0, 80), (0, 100), (0, 110), (0, 0))
_ONES = 0x01010101


def _mask_kernel(pads_ref, out_ref):
    b = pl.program_id(0)
    pe = pads_ref[b, 1]

    nh = _IMG_BEGIN // 4
    r_h = jax.lax.broadcasted_iota(jnp.int32, (nh, 1), 0) * 4
    kv_h = jax.lax.broadcasted_iota(jnp.int32, (nh, _S), 1)
    w_h = jnp.zeros((nh, _S), jnp.int32)
    for k in range(4):
        qk = r_h + k
        lo = jnp.minimum(qk, pe)
        m = (kv_h >= lo) & (kv_h <= qk)
        w_h = w_h | jnp.where(m, jnp.int32(1 << (8 * k)), 0)
    out_ref[0, 0:_IMG_BEGIN, :] = pltpu.bitcast(w_h, jnp.uint8)

    nc = (_IMG_END - _IMG_BEGIN) // 4
    kv_c = jax.lax.broadcasted_iota(jnp.int32, (nc, _S), 1)
    w_c = jnp.where((kv_c >= pe) & (kv_c < _IMG_END), jnp.int32(_ONES), 0)
    out_ref[0, _IMG_BEGIN:_IMG_END, :] = pltpu.bitcast(w_c, jnp.uint8)

    nt = (_S - _IMG_END) // 4
    r_t = _IMG_END + jax.lax.broadcasted_iota(jnp.int32, (nt, 1), 0) * 4
    kv_t = jax.lax.broadcasted_iota(jnp.int32, (nt, _S), 1)
    d = kv_t - r_t
    s = jnp.minimum(jnp.maximum(d, 0), 3) * 8
    pat = jax.lax.shift_left(jnp.int32(_ONES), s)
    w_t = jnp.where((d <= 3) & (kv_t >= pe), pat, 0)
    out_ref[0, _IMG_END:_S, :] = pltpu.bitcast(w_t, jnp.uint8)


def kernel(sequence):
    B, S = sequence.shape
    pads = jnp.asarray(_PAD_BEGIN_ENDS, dtype=jnp.int32)
    packed = pl.pallas_call(
        _mask_kernel,
        grid=(B,),
        in_specs=[pl.BlockSpec(memory_space=pltpu.SMEM)],
        out_specs=pl.BlockSpec((1, S, S), lambda b: (b, 0, 0)),
        out_shape=jax.ShapeDtypeStruct((B, S, S), jnp.uint8),
        compiler_params=pltpu.CompilerParams(
            dimension_semantics=("parallel",),
        ),
    )(pads)
    return packed.astype(jnp.bool_)
